# baseline (speedup 1.0000x reference)
import functools

import jax
import jax.numpy as jnp
from jax.experimental import pallas as pl


def _tc_body(x_ref, w_ref, out_ref):
    xv = x_ref[...]
    w1 = w_ref[1, 0]
    w2 = w_ref[2, 0]
    w3 = w_ref[3, 0]
    w4 = w_ref[4, 0]
    res = jnp.where(
        xv == 1,
        w1,
        jnp.where(xv == 2, w2, jnp.where(xv == 3, w3, jnp.where(xv == 4, w4, 0.0))),
    )
    out_ref[...] = res.T


@functools.partial(jax.jit, static_argnames=("rows", "cols"))
def _lookup(x, weight, rows, cols):
    rt = 512
    tc_fn = pl.pallas_call(
        _tc_body,
        grid=(rows // rt, 2),
        in_specs=[
            pl.BlockSpec((rt, 128), lambda r, c: (r, c)),
            pl.BlockSpec((weight.shape[0], 1), lambda r, c: (0, 0)),
        ],
        out_specs=pl.BlockSpec((128, rt), lambda r, c: (c, r)),
        out_shape=jax.ShapeDtypeStruct((cols, rows), jnp.float32),
    )
    return tc_fn(x, weight)


def kernel(x, weight):
    b, s = x.shape
    out_t = _lookup(x.astype(jnp.int32), weight.astype(jnp.float32), b, s)
    return out_t.T.reshape(b, s, 1)


# speedup vs baseline: 1.3186x; 1.3186x over previous
import functools

import jax
import jax.numpy as jnp
from jax import lax
from jax.experimental import pallas as pl
from jax.experimental.pallas import tpu as pltpu
from jax.experimental.pallas import tpu_sc as plsc

_NC = 2
_NS = 16
_NW = _NC * _NS
_L = 16

_SC_COLS = 128
_TC_COL_BLK = 128


def _sc_body(rows_w, rows_c, x_hbm, w_hbm, out_hbm, tbl_v, i0, i1, o0, o1, si0, si1, so0, so1):
    wid = lax.axis_index("s") * _NC + lax.axis_index("c")
    base = wid * rows_w
    pltpu.sync_copy(w_hbm, tbl_v)

    idx = (i0, i1)
    out = (o0, o1)
    sin = (si0, si1)
    sout = (so0, so1)
    n = rows_w // rows_c
    in_h = [None] * n
    out_h = [None] * n

    def rbof(c):
        return pl.multiple_of(base + c * rows_c, 8)

    def start_in(c):
        h = pltpu.make_async_copy(
            x_hbm.at[pl.ds(rbof(c), rows_c), pl.ds(0, _SC_COLS)],
            idx[c % 2],
            sin[c % 2],
        )
        h.start()
        in_h[c] = h

    def start_out(c):
        h = pltpu.make_async_copy(
            out[c % 2],
            out_hbm.at[pl.ds(rbof(c), rows_c), pl.ds(0, _SC_COLS)],
            sout[c % 2],
        )
        h.start()
        out_h[c] = h

    start_in(0)
    for c in range(n):
        bi = c % 2
        if c + 1 < n:
            start_in(c + 1)
        in_h[c].wait()
        if c >= 2:
            out_h[c - 2].wait()

        @plsc.parallel_loop(0, rows_c, 1, unroll=2)
        def row_step(r, _iv=idx[bi], _ov=out[bi]):
            irow = _iv.at[r]
            orow = _ov.at[r]
            for o in range(0, _SC_COLS, _L):
                orow[pl.ds(o, _L)] = plsc.load_gather(tbl_v, [irow[pl.ds(o, _L)]])

        start_out(c)
    out_h[n - 2].wait()
    out_h[n - 1].wait()


def _tc_body(x_ref, w_ref, part_ref, out_ref):
    del part_ref
    xv = x_ref[...]
    w1 = w_ref[1, 0]
    w2 = w_ref[2, 0]
    w3 = w_ref[3, 0]
    w4 = w_ref[4, 0]
    out_ref[...] = jnp.where(
        xv == 1,
        w1,
        jnp.where(xv == 2, w2, jnp.where(xv == 3, w3, jnp.where(xv == 4, w4, 0.0))),
    )


@functools.partial(jax.jit, static_argnames=("rows", "cols"))
def _lookup(x, weight, w_pad, rows, cols):
    rows_w = rows // _NW
    rows_c = 128
    sc_fn = pl.kernel(
        functools.partial(_sc_body, rows_w, rows_c),
        out_type=jax.ShapeDtypeStruct((rows, cols), jnp.float32),
        mesh=plsc.VectorSubcoreMesh(core_axis_name="c", subcore_axis_name="s"),
        compiler_params=pltpu.CompilerParams(
            needs_layout_passes=False, use_tc_tiling_on_sc=True
        ),
        scratch_types=[
            pltpu.VMEM((_L,), jnp.float32),
            pltpu.VMEM((rows_c, _SC_COLS), jnp.int32),
            pltpu.VMEM((rows_c, _SC_COLS), jnp.int32),
            pltpu.VMEM((rows_c, _SC_COLS), jnp.float32),
            pltpu.VMEM((rows_c, _SC_COLS), jnp.float32),
            pltpu.SemaphoreType.DMA,
            pltpu.SemaphoreType.DMA,
            pltpu.SemaphoreType.DMA,
            pltpu.SemaphoreType.DMA,
        ],
    )
    part = sc_fn(x, w_pad)

    rt = 4096
    tc_fn = pl.pallas_call(
        _tc_body,
        grid=(rows // rt,),
        in_specs=[
            pl.BlockSpec((rt, _TC_COL_BLK), lambda r: (r, 1)),
            pl.BlockSpec((weight.shape[0], 1), lambda r: (0, 0)),
            pl.BlockSpec(memory_space=pl.ANY),
        ],
        out_specs=pl.BlockSpec((rt, _TC_COL_BLK), lambda r: (r, 1)),
        out_shape=jax.ShapeDtypeStruct((rows, cols), jnp.float32),
        input_output_aliases={2: 0},
    )
    return tc_fn(x, weight, part)


def kernel(x, weight):
    b, s = x.shape
    w_pad = jnp.zeros((_L,), jnp.float32).at[: weight.shape[0]].set(
        weight.reshape(-1).astype(jnp.float32)
    )
    out = _lookup(x.astype(jnp.int32), weight.astype(jnp.float32), w_pad, b, s)
    return out.reshape(b, s, 1)


# speedup vs baseline: 1.3217x; 1.0024x over previous
import functools

import jax
import jax.numpy as jnp
from jax import lax
from jax.experimental import pallas as pl
from jax.experimental.pallas import tpu as pltpu
from jax.experimental.pallas import tpu_sc as plsc

_NC = 2
_NS = 16
_NW = _NC * _NS
_L = 16

_SC_COLS = 128
_TC_COL_BLK = 128


def _sc_body(rows_w, rows_c, x_hbm, w_hbm, out_hbm, tbl_v, i0, i1, o0, o1, si0, si1, so0, so1):
    wid = lax.axis_index("s") * _NC + lax.axis_index("c")
    base = wid * rows_w
    pltpu.sync_copy(w_hbm, tbl_v)

    idx = (i0, i1)
    out = (o0, o1)
    sin = (si0, si1)
    sout = (so0, so1)
    n = rows_w // rows_c
    in_h = [None] * n
    out_h = [None] * n

    def rbof(c):
        return pl.multiple_of(base + c * rows_c, 8)

    def start_in(c):
        h = pltpu.make_async_copy(
            x_hbm.at[pl.ds(rbof(c), rows_c), pl.ds(0, _SC_COLS)],
            idx[c % 2],
            sin[c % 2],
        )
        h.start()
        in_h[c] = h

    def start_out(c):
        h = pltpu.make_async_copy(
            out[c % 2],
            out_hbm.at[pl.ds(rbof(c), rows_c), pl.ds(0, _SC_COLS)],
            sout[c % 2],
        )
        h.start()
        out_h[c] = h

    start_in(0)
    for c in range(n):
        bi = c % 2
        if c + 1 < n:
            start_in(c + 1)
        in_h[c].wait()
        if c >= 2:
            out_h[c - 2].wait()

        @plsc.parallel_loop(0, rows_c, 1, unroll=4)
        def row_step(r, _iv=idx[bi], _ov=out[bi]):
            irow = _iv.at[r]
            orow = _ov.at[r]
            for o in range(0, _SC_COLS, _L):
                orow[pl.ds(o, _L)] = plsc.load_gather(tbl_v, [irow[pl.ds(o, _L)]])

        start_out(c)
    out_h[n - 2].wait()
    out_h[n - 1].wait()


def _tc_body(x_ref, w_ref, part_ref, out_ref):
    del part_ref
    xv = x_ref[...]
    w1 = w_ref[1, 0]
    w2 = w_ref[2, 0]
    w3 = w_ref[3, 0]
    w4 = w_ref[4, 0]
    out_ref[...] = jnp.where(
        xv == 1,
        w1,
        jnp.where(xv == 2, w2, jnp.where(xv == 3, w3, jnp.where(xv == 4, w4, 0.0))),
    )


@functools.partial(jax.jit, static_argnames=("rows", "cols"))
def _lookup(x, weight, w_pad, rows, cols):
    rows_w = rows // _NW
    rows_c = 128
    sc_fn = pl.kernel(
        functools.partial(_sc_body, rows_w, rows_c),
        out_type=jax.ShapeDtypeStruct((rows, cols), jnp.float32),
        mesh=plsc.VectorSubcoreMesh(core_axis_name="c", subcore_axis_name="s"),
        compiler_params=pltpu.CompilerParams(
            needs_layout_passes=False, use_tc_tiling_on_sc=True
        ),
        scratch_types=[
            pltpu.VMEM((_L,), jnp.float32),
            pltpu.VMEM((rows_c, _SC_COLS), jnp.int32),
            pltpu.VMEM((rows_c, _SC_COLS), jnp.int32),
            pltpu.VMEM((rows_c, _SC_COLS), jnp.float32),
            pltpu.VMEM((rows_c, _SC_COLS), jnp.float32),
            pltpu.SemaphoreType.DMA,
            pltpu.SemaphoreType.DMA,
            pltpu.SemaphoreType.DMA,
            pltpu.SemaphoreType.DMA,
        ],
    )
    part = sc_fn(x, w_pad)

    rt = 4096
    tc_fn = pl.pallas_call(
        _tc_body,
        grid=(rows // rt,),
        in_specs=[
            pl.BlockSpec((rt, _TC_COL_BLK), lambda r: (r, 1)),
            pl.BlockSpec((weight.shape[0], 1), lambda r: (0, 0)),
            pl.BlockSpec(memory_space=pl.ANY),
        ],
        out_specs=pl.BlockSpec((rt, _TC_COL_BLK), lambda r: (r, 1)),
        out_shape=jax.ShapeDtypeStruct((rows, cols), jnp.float32),
        input_output_aliases={2: 0},
    )
    return tc_fn(x, weight, part)


def kernel(x, weight):
    b, s = x.shape
    w_pad = jnp.zeros((_L,), jnp.float32).at[: weight.shape[0]].set(
        weight.reshape(-1).astype(jnp.float32)
    )
    out = _lookup(x.astype(jnp.int32), weight.astype(jnp.float32), w_pad, b, s)
    return out.reshape(b, s, 1)
